# Initial kernel scaffold; baseline (speedup 1.0000x reference)
import jax
import jax.numpy as jnp
from jax.experimental import pallas as pl


def kernel(match, mask):
    raise NotImplementedError("write your pallas kernel here")


# speedup vs baseline: 1.2119x; 1.2119x over previous
import functools

import jax
import jax.numpy as jnp
from jax import lax
from jax.experimental import pallas as pl
from jax.experimental.pallas import tpu as pltpu
from jax.experimental.pallas import tpu_sc as plsc

_CHECK_NUM = 6000
_RATIO = 0.2

_NC = 2
_NS = 16
_NW = _NC * _NS
_CHUNKS = 24
_CW = 128


def _sc_gather(flat_m, idx):
  mesh = plsc.VectorSubcoreMesh(core_axis_name="c", subcore_axis_name="s")

  @functools.partial(
      pl.kernel,
      out_type=jax.ShapeDtypeStruct((_NW, _CHUNKS, _CW), jnp.float32),
      mesh=mesh,
      scratch_types=[
          pltpu.VMEM((_CHUNKS, _CW), jnp.int32),
          pltpu.VMEM((_CHUNKS, _CW), jnp.float32),
          pltpu.SemaphoreType.DMA,
      ],
  )
  def k(m_hbm, idx_hbm, out_hbm, idx_v, rows_v, sem):
    cid = lax.axis_index("c")
    sid = lax.axis_index("s")
    wid = sid * _NC + cid
    pltpu.sync_copy(idx_hbm.at[wid], idx_v)
    half = _CHUNKS // 2
    for base in (0, half):
      descs = [
          pltpu.async_copy(m_hbm.at[idx_v.at[base + j]], rows_v.at[base + j], sem)
          for j in range(half)
      ]
      for d in descs:
        d.wait()
    pltpu.sync_copy(rows_v, out_hbm.at[wid])

  return k(flat_m, idx)


def kernel(match, mask):
  b = match.shape[0]
  n = match.shape[2] * match.shape[3]
  m = match.reshape(b, 4, n)
  msk = mask.reshape(b, n)
  k = int(_RATIO * n)

  _, indices = jax.lax.top_k(msk, k)
  rand_int = jax.random.randint(jax.random.key(1), (_CHECK_NUM,), 0, k)
  j_idx = jnp.take(indices, rand_int, axis=1)

  bc_off = (jnp.arange(b)[:, None, None] * 4 + jnp.arange(4)[None, :, None]) * n
  flat_idx = (bc_off + j_idx[:, None, :]).reshape(-1)

  total = _NW * _CHUNKS * _CW
  pad = total - flat_idx.shape[0]
  pad_idx = (jnp.arange(pad, dtype=jnp.int32) * 64) % (b * 4 * n)
  idx_all = jnp.concatenate([flat_idx.astype(jnp.int32), pad_idx])
  idx_all = idx_all.reshape(_NW, _CHUNKS, _CW)

  out = _sc_gather(m.reshape(-1), idx_all)
  return out.reshape(-1)[: b * 4 * _CHECK_NUM].reshape(b, 4, _CHECK_NUM)


# speedup vs baseline: 2.8282x; 2.3337x over previous
import functools

import jax
import jax.numpy as jnp
from jax import lax
from jax.experimental import pallas as pl
from jax.experimental.pallas import tpu as pltpu
from jax.experimental.pallas import tpu_sc as plsc

_CHECK_NUM = 6000
_RATIO = 0.2

_N = 262144
_NB = 4
_NSUB = 16
_SH = _N // _NSUB
_BINS = 16384
_BLK = 2048
_RP = 6144
_PW = _RP // _NSUB
_CAPS = 262144
_TRASH = _CAPS
_CARR = _CAPS + 4096
_PADKEY = 0x7FFFFFFF
_PADIX = 0x3FFFFFFF
_I16 = None


def _ds(off, size):
  if isinstance(off, int):
    return pl.ds(off, size)
  return pl.ds(pl.multiple_of(off, 8), size)


def _gather16(x, idx):
  dn = lax.GatherDimensionNumbers(
      offset_dims=(), collapsed_slice_dims=(0,), start_index_map=(0,))
  return lax.gather(x, idx[:, None], dn, (1,),
                    mode=lax.GatherScatterMode.PROMISE_IN_BOUNDS)


def _prefix_incl(x):
  i16 = lax.iota(jnp.int32, 16)
  acc = x
  for d in (1, 2, 4, 8):
    s = _gather16(acc, jnp.maximum(i16 - d, 0))
    acc = acc + jnp.where(i16 >= d, s, 0)
  return acc


def _hmin16(x):
  i16 = lax.iota(jnp.int32, 16)
  acc = x
  for d in (1, 2, 4, 8):
    s = _gather16(acc, jnp.minimum(i16 + d, 15))
    acc = jnp.minimum(acc, jnp.where(i16 < 16 - d, s, acc))
  return acc


def _dynlane(vec, s):
  i16 = lax.iota(jnp.int32, 16)
  return _gather16(vec, (i16 + s) & 15)[0]


def _bsel(cond_scalar, a, b):
  return jnp.where(jnp.broadcast_to(cond_scalar, (16,)), a, b)


def _lex_gt(ak, ai, bk, bi):
  return (ak > bk) | ((ak == bk) & (ai > bi))


def _sc_topk_sample(mask_flat, match_flat, rand, cfg):
  mesh = plsc.VectorSubcoreMesh(core_axis_name="c", subcore_axis_name="s")

  @functools.partial(
      pl.kernel,
      out_type=(jax.ShapeDtypeStruct((_NB * 4 * _RP,), jnp.float32),
                jax.ShapeDtypeStruct((2 * 65536 + 32 * _PW,), jnp.int32)),
      mesh=mesh,
      scratch_types=[
          pltpu.VMEM((_BLK,), jnp.float32),
          pltpu.VMEM((_BLK,), jnp.int32),
          pltpu.VMEM((_BLK,), jnp.int32),
          pltpu.VMEM((_BLK,), jnp.int32),
          pltpu.VMEM((_CAPS // _NSUB,), jnp.int32),
          pltpu.VMEM((_CAPS // _NSUB,), jnp.int32),
          pltpu.VMEM((_BLK,), jnp.int32),
          pltpu.VMEM((_BLK,), jnp.int32),
          pltpu.VMEM((_BINS,), jnp.int32),
          pltpu.VMEM((128,), jnp.int32),
          pltpu.VMEM((128,), jnp.int32),
          pltpu.VMEM((128,), jnp.int32),
          pltpu.VMEM((128,), jnp.int32),
          pltpu.VMEM((128,), jnp.int32),
          pltpu.VMEM((_PW,), jnp.int32),
          pltpu.VMEM((_PW,), jnp.int32),
          pltpu.VMEM((_PW,), jnp.int32),
          pltpu.VMEM((_PW,), jnp.float32),
          pltpu.VMEM_SHARED((_CARR,), jnp.int32),
          pltpu.VMEM_SHARED((_CARR,), jnp.int32),
          pltpu.VMEM_SHARED((_BINS,), jnp.int32),
          pltpu.VMEM_SHARED((256,), jnp.int32),
          pltpu.SemaphoreType.DMA,
      ],
  )
  def k(mask_hbm, match_hbm, rand_hbm, cfg_hbm, out_hbm, ixh_hbm,
        fbuf, kv, iv, pv, ku_t, ix_t, pk, pi, hb, mbv, sidx, sval,
        gi, gv, rv, jv, av, fv, ku_c, ix_c, hist, meta, sem):
    cid = lax.axis_index("c")
    sid = lax.axis_index("s")
    i16 = lax.iota(jnp.int32, 16)
    zeros = jnp.zeros((16,), jnp.int32)

    pltpu.sync_copy(cfg_hbm, mbv)
    r_max = mbv[0:16][0]

    def batch_body(bi, carry0):
      batch = cid * 2 + bi
      moff = batch * _N
      shoff = sid * _SH

      def z(l, c):
        kv[_ds(l * 16, 16)] = zeros
        return c
      lax.fori_loop(0, _BLK // 16, z, jnp.int32(0))
      pltpu.sync_copy(kv.at[_ds(0, 1024)],
                      hist.at[_ds(sid * 1024, 1024)])

      @pl.when(sid == 0)
      def _():
        pltpu.sync_copy(kv.at[_ds(0, 256)], meta)
      plsc.subcore_barrier()

      def o(l, c):
        iv[_ds(l * 16, 16)] = jnp.full((16,), 1, jnp.int32)
        return c
      lax.fori_loop(0, _BLK // 16, o, jnp.int32(0))

      def blk1(t, c):
        pltpu.sync_copy(mask_hbm.at[_ds(moff + shoff + t * _BLK, _BLK)],
                        fbuf)
        def inner(l, c2):
          v = fbuf[_ds(l * 16, 16)]
          basc = jnp.minimum(jnp.maximum(v * 16384.0, 0.0), 16383.0)
          binv = 16383 - basc.astype(jnp.int32)
          pv[_ds(l * 16, 16)] = binv
          return c2
        lax.fori_loop(0, _BLK // 16, inner, jnp.int32(0))
        pltpu.sync_copy(iv, hist.at[pv], add=True)
        return c
      lax.fori_loop(0, _SH // _BLK, blk1, jnp.int32(0))
      plsc.subcore_barrier()

      @pl.when(sid == 0)
      def _():
        pltpu.sync_copy(hist, hb)
        def scan(j, carry):
          cum, bmax = carry
          h = hb[_ds(j * 16, 16)]
          inc = _prefix_incl(h) + jnp.broadcast_to(cum, (16,))
          newcum = inc[15]
          exceed = inc > jnp.broadcast_to(r_max, (16,))
          lane = _hmin16(jnp.where(exceed, i16, 16))[0]
          cand = j * 16 + lane
          found_now = jnp.logical_and(cum <= r_max, newcum > r_max)
          bmax = lax.select(found_now, cand, bmax)
          return (newcum, bmax)
        _, bmax = lax.fori_loop(0, _BINS // 16, scan,
                                (jnp.int32(0), jnp.int32(_BINS - 1)))
        def wsm(l, c):
          jj = l * 16 + i16
          sidx[_ds(l * 16, 16)] = jnp.where(jj == 0, 0, 128 + jj)
          sval[_ds(l * 16, 16)] = jnp.broadcast_to(bmax, (16,))
          return c
        lax.fori_loop(0, 8, wsm, jnp.int32(0))
        pltpu.sync_copy(sval, meta.at[sidx])
      plsc.subcore_barrier()

      pltpu.sync_copy(meta.at[_ds(0, 128)], mbv)
      b_max = mbv[0:16][0]

      def blk3(t, cvec):
        pltpu.sync_copy(mask_hbm.at[_ds(moff + shoff + t * _BLK, _BLK)],
                        fbuf)
        def inner(l, cv):
          v = fbuf[_ds(l * 16, 16)]
          basc = jnp.minimum(jnp.maximum(v * 16384.0, 0.0), 16383.0)
          binv = 16383 - basc.astype(jnp.int32)
          return cv + jnp.where(binv <= b_max, 1, 0)
        return lax.fori_loop(0, _BLK // 16, inner, cvec)
      cvec = lax.fori_loop(0, _SH // _BLK, blk3, zeros)
      cnt = _prefix_incl(cvec)[15]
      def wcm(l, c):
        jj = l * 16 + i16
        sidx[_ds(l * 16, 16)] = jnp.where(jj == 0, 16 + sid,
                                          128 + ((sid * 8 + jj) & 127))
        sval[_ds(l * 16, 16)] = jnp.broadcast_to(cnt, (16,))
        return c
      lax.fori_loop(0, 8, wcm, jnp.int32(0))
      pltpu.sync_copy(sval, meta.at[sidx])
      plsc.subcore_barrier()

      pltpu.sync_copy(meta.at[_ds(0, 128)], mbv)
      call = mbv[16:32]
      c128 = (call + 127) & (-128)
      pfx = _prefix_incl(c128)
      m128 = pfx[15]
      base_w = _dynlane(pfx - c128, sid)
      cnt_w = _dynlane(call, sid)
      c128_w = _dynlane(c128, sid)

      extra = (jnp.where(m128 > 65536, 1, 0)
               + jnp.where(m128 > 131072, 1, 0))
      pmax = 16 + extra
      n_dyn = lax.shift_left(jnp.int32(65536), extra)
      cs = n_dyn // _NSUB
      logl = pmax - 4
      trips = cs // _BLK

      def blk4(t, run):
        pltpu.sync_copy(mask_hbm.at[_ds(moff + shoff + t * _BLK, _BLK)],
                        fbuf)
        def inner(l, r):
          v = fbuf[_ds(l * 16, 16)]
          u = lax.bitcast_convert_type(v, jnp.int32)
          kms = jnp.where(u >= 0, u, u ^ 0x7FFFFFFF)
          kmd = ~kms
          basc = jnp.minimum(jnp.maximum(v * 16384.0, 0.0), 16383.0)
          binv = 16383 - basc.astype(jnp.int32)
          m = binv <= b_max
          mi = jnp.where(m, 1, 0)
          pfxi = _prefix_incl(mi)
          excl = pfxi - mi
          loc = t * _BLK + l * 16 + i16
          gidx = shoff + loc
          pos = jnp.where(m, base_w + r + excl,
                          _TRASH + (loc & (_BLK - 1)))
          kv[_ds(l * 16, 16)] = kmd
          iv[_ds(l * 16, 16)] = gidx
          pv[_ds(l * 16, 16)] = pos
          return r + pfxi[15]
        run = lax.fori_loop(0, _BLK // 16, inner, run)
        pltpu.sync_copy(kv, ku_c.at[pv])
        pltpu.sync_copy(iv, ix_c.at[pv])
        return run
      lax.fori_loop(0, _SH // _BLK, blk4, jnp.int32(0))

      gap = c128_w - cnt_w
      def gfill(l, c):
        j = l * 16 + i16
        gi[_ds(l * 16, 16)] = jnp.where(j < gap, base_w + cnt_w + j,
                                          _TRASH + j)
        gv[_ds(l * 16, 16)] = jnp.full((16,), _PADKEY, jnp.int32)
        return c
      lax.fori_loop(0, 8, gfill, jnp.int32(0))
      pltpu.sync_copy(gv, ku_c.at[gi])
      def gfill2(l, c):
        gv[_ds(l * 16, 16)] = jnp.full((16,), _PADIX, jnp.int32)
        return c
      lax.fori_loop(0, 8, gfill2, jnp.int32(0))
      pltpu.sync_copy(gv, ix_c.at[gi])

      def pfill(l, c):
        kv[_ds(l * 16, 16)] = jnp.full((16,), _PADKEY, jnp.int32)
        iv[_ds(l * 16, 16)] = jnp.full((16,), _PADIX, jnp.int32)
        return c
      lax.fori_loop(0, 8, pfill, jnp.int32(0))
      nfill = (n_dyn - m128) // 128
      mytrips = (nfill - sid + _NSUB - 1) // _NSUB
      def tfill(t, c):
        start = m128 + (sid + t * _NSUB) * 128
        pltpu.sync_copy(kv.at[_ds(0, 128)], ku_c.at[_ds(start, 128)])
        pltpu.sync_copy(iv.at[_ds(0, 128)], ix_c.at[_ds(start, 128)])
        return c
      lax.fori_loop(0, jnp.maximum(mytrips, 0), tfill, jnp.int32(0))
      plsc.subcore_barrier()

      def ld(t, c):
        pltpu.sync_copy(ku_c.at[_ds(sid * cs + t * _BLK, _BLK)],
                        kv)
        pltpu.sync_copy(ix_c.at[_ds(sid * cs + t * _BLK, _BLK)],
                        iv)
        def cp(l, c2):
          ku_t[_ds(t * _BLK + l * 16, 16)] = kv[_ds(l * 16, 16)]
          ix_t[_ds(t * _BLK + l * 16, 16)] = iv[_ds(l * 16, 16)]
          return c2
        lax.fori_loop(0, _BLK // 16, cp, jnp.int32(0))
        return c
      lax.fori_loop(0, trips, ld, jnp.int32(0))

      def publish(t, c):
        def cp(l, c2):
          kv[_ds(l * 16, 16)] = ku_t[_ds(t * _BLK + l * 16, 16)]
          iv[_ds(l * 16, 16)] = ix_t[_ds(t * _BLK + l * 16, 16)]
          return c2
        lax.fori_loop(0, _BLK // 16, cp, jnp.int32(0))
        pltpu.sync_copy(kv, ku_c.at[_ds(sid * cs + t * _BLK, _BLK)])
        pltpu.sync_copy(iv, ix_c.at[_ds(sid * cs + t * _BLK, _BLK)])
        return c

      def cross_stage(p, dlog):
        lax.fori_loop(0, trips, publish, jnp.int32(0))
        plsc.subcore_barrier()
        dchunks = lax.shift_left(jnp.int32(1), dlog - logl)
        partner = lax.bitwise_xor(sid, dchunks)
        kh_i = jnp.where((sid & dchunks) > 0, 1, 0)
        asc_i = 1 - (lax.shift_right_logical(sid * cs, p) & 1)
        tm_i = jnp.where(kh_i == asc_i, 1, 0)
        def cblk(t, c):
          pltpu.sync_copy(ku_c.at[_ds(partner * cs + t * _BLK, _BLK)], pk)
          pltpu.sync_copy(ix_c.at[_ds(partner * cs + t * _BLK, _BLK)], pi)
          def ce(l, c2):
            off = t * _BLK + l * 16
            a = ku_t[_ds(off, 16)]
            ai = ix_t[_ds(off, 16)]
            b = pk[_ds(l * 16, 16)]
            bi_ = pi[_ds(l * 16, 16)]
            m_i = jnp.where(_lex_gt(a, ai, b, bi_), 1, 0)
            tb = (m_i ^ jnp.broadcast_to(tm_i, (16,))) != 0
            ku_t[_ds(off, 16)] = jnp.where(tb, b, a)
            ix_t[_ds(off, 16)] = jnp.where(tb, bi_, ai)
            return c2
          lax.fori_loop(0, _BLK // 16, ce, jnp.int32(0))
          return c
        lax.fori_loop(0, trips, cblk, jnp.int32(0))
        plsc.subcore_barrier()

      def local_big_stage(p, dlog):
        d = lax.shift_left(jnp.int32(1), dlog)
        nblk = lax.shift_right_logical(cs, dlog + 1)
        vpb = lax.shift_right_logical(d, 4)
        def bb(bidx, c):
          boff = lax.shift_left(bidx, dlog + 1)
          g0 = sid * cs + boff
          desc_i = lax.shift_right_logical(g0, p) & 1
          def qq(q, c2):
            off = boff + q * 16
            a = ku_t[_ds(off, 16)]
            ai = ix_t[_ds(off, 16)]
            b = ku_t[_ds(off + d, 16)]
            bi_ = ix_t[_ds(off + d, 16)]
            m_i = jnp.where(_lex_gt(a, ai, b, bi_), 1, 0)
            sw = (m_i ^ jnp.broadcast_to(desc_i, (16,))) != 0
            ku_t[_ds(off, 16)] = jnp.where(sw, b, a)
            ix_t[_ds(off, 16)] = jnp.where(sw, bi_, ai)
            ku_t[_ds(off + d, 16)] = jnp.where(sw, a, b)
            ix_t[_ds(off + d, 16)] = jnp.where(sw, ai, bi_)
            return c2
          lax.fori_loop(0, vpb, qq, jnp.int32(0))
          return c
        lax.fori_loop(0, nblk, bb, jnp.int32(0))

      def invreg_stage(p, d):
        def qq(q, c):
          i16b = lax.iota(jnp.int32, 16)
          xor_idx = lax.bitwise_xor(i16b, d)
          il_i = jnp.where((i16b & d) == 0, 1, 0)
          off = q * 16
          g = sid * cs + off + i16b
          x = ku_t[_ds(off, 16)]
          xi = ix_t[_ds(off, 16)]
          px = _gather16(x, xor_idx)
          pxi = _gather16(xi, xor_idx)
          asc_i = 1 - (lax.shift_right_logical(
              g, jnp.broadcast_to(p, (16,))) & 1)
          m_i = jnp.where(_lex_gt(x, xi, px, pxi), 1, 0)
          ks_i = jnp.where(il_i == asc_i, 1, 0)
          tp = (m_i ^ ks_i ^ 1) != 0
          ku_t[_ds(off, 16)] = jnp.where(tp, px, x)
          ix_t[_ds(off, 16)] = jnp.where(tp, pxi, xi)
          return c
        lax.fori_loop(0, cs // 16, qq, jnp.int32(0))

      def phase_body(p, c):
        def stage_body(t, c2):
          dlog = p - 1 - t
          @pl.when(dlog >= logl)
          def _():
            cross_stage(p, dlog)
          @pl.when(dlog < logl)
          def _():
            local_big_stage(p, dlog)
          return c2
        lax.fori_loop(0, jnp.maximum(p - 4, 0), stage_body, jnp.int32(0))
        for dlog in (3, 2, 1, 0):
          @pl.when(p > dlog)
          def _():
            invreg_stage(p, 1 << dlog)
        return c
      lax.fori_loop(1, pmax + 1, phase_body, jnp.int32(0))

      def fpub(t, c):
        def cp(l, c2):
          iv[_ds(l * 16, 16)] = ix_t[_ds(t * _BLK + l * 16, 16)]
          return c2
        lax.fori_loop(0, _BLK // 16, cp, jnp.int32(0))
        pos = sid * cs + t * _BLK
        @pl.when(pos < 65536)
        def _():
          pltpu.sync_copy(iv, ixh_hbm.at[_ds(cid * 65536 + pos, _BLK)])
        return c
      lax.fori_loop(0, trips, fpub, jnp.int32(0))
      plsc.subcore_barrier()

      pltpu.sync_copy(rand_hbm.at[_ds(sid * _PW, _PW)], rv)
      ixview = ixh_hbm.at[_ds(cid * 65536, 65536)]
      for jj in range(_PW // 128):
        pltpu.sync_copy(ixview.at[rv.at[_ds(jj * 128, 128)]],
                        jv.at[_ds(jj * 128, 128)])
      jslot = 2 * 65536 + (cid * _NSUB + sid) * _PW
      pltpu.sync_copy(jv, ixh_hbm.at[_ds(jslot, _PW)])
      pltpu.sync_copy(ixh_hbm.at[_ds(jslot, _PW)], av)
      for ch in range(4):
        chview = match_hbm.at[_ds((batch * 4 + ch) * _N, _N)]
        for jj in range(_PW // 128):
          pltpu.sync_copy(chview.at[av.at[_ds(jj * 128, 128)]],
                          fv.at[_ds(jj * 128, 128)])
        pltpu.sync_copy(
            fv, out_hbm.at[_ds((batch * 4 + ch) * _RP + sid * _PW, _PW)])
      plsc.subcore_barrier()
      return carry0

    lax.fori_loop(0, 2, batch_body, jnp.int32(0))

  return k(mask_flat, match_flat, rand, cfg)


def kernel(match, mask):
  b = match.shape[0]
  n = match.shape[2] * match.shape[3]
  k = int(_RATIO * n)
  m_flat = match.reshape(-1)
  mask_flat = mask.reshape(-1)

  rand_int = jax.random.randint(jax.random.key(1), (_CHECK_NUM,), 0, k)
  rand_pad = jnp.concatenate(
      [rand_int.astype(jnp.int32),
       jnp.zeros((_RP - _CHECK_NUM,), jnp.int32)])
  r_max = jnp.max(rand_int).astype(jnp.int32)
  cfg = jnp.concatenate([r_max[None], jnp.zeros((127,), jnp.int32)])

  outf, _ = _sc_topk_sample(mask_flat, m_flat, rand_pad, cfg)
  return outf.reshape(b, 4, _RP)[:, :, :_CHECK_NUM]


# speedup vs baseline: 5.9120x; 2.0904x over previous
import functools

import jax
import jax.numpy as jnp
from jax import lax
from jax.experimental import pallas as pl
from jax.experimental.pallas import tpu as pltpu
from jax.experimental.pallas import tpu_sc as plsc

_CHECK_NUM = 6000
_RATIO = 0.2

_N = 262144
_NB = 4
_NSUB = 16
_SH = _N // _NSUB
_BINS = 16384
_BLK = 2048
_RP = 6144
_PW = _RP // _NSUB
_CAPS = 262144
_TRASH = _CAPS
_CARR = _CAPS + 4096
_PADKEY = 0x7FFFFFFF
_PADIX = 0x3FFFFFFF
_I16 = None


def _ds(off, size):
  if isinstance(off, int):
    return pl.ds(off, size)
  return pl.ds(pl.multiple_of(off, 8), size)


def _gather16(x, idx):
  dn = lax.GatherDimensionNumbers(
      offset_dims=(), collapsed_slice_dims=(0,), start_index_map=(0,))
  return lax.gather(x, idx[:, None], dn, (1,),
                    mode=lax.GatherScatterMode.PROMISE_IN_BOUNDS)


def _prefix_incl(x):
  i16 = lax.iota(jnp.int32, 16)
  acc = x
  for d in (1, 2, 4, 8):
    s = _gather16(acc, jnp.maximum(i16 - d, 0))
    acc = acc + jnp.where(i16 >= d, s, 0)
  return acc


def _hmin16(x):
  i16 = lax.iota(jnp.int32, 16)
  acc = x
  for d in (1, 2, 4, 8):
    s = _gather16(acc, jnp.minimum(i16 + d, 15))
    acc = jnp.minimum(acc, jnp.where(i16 < 16 - d, s, acc))
  return acc


def _dynlane(vec, s):
  i16 = lax.iota(jnp.int32, 16)
  return _gather16(vec, (i16 + s) & 15)[0]


def _bsel(cond_scalar, a, b):
  return jnp.where(jnp.broadcast_to(cond_scalar, (16,)), a, b)


def _lex_gt(ak, ai, bk, bi):
  return (ak > bk) | ((ak == bk) & (ai > bi))


def _sc_topk_sample(mask_flat, match_flat, rand, cfg):
  mesh = plsc.VectorSubcoreMesh(core_axis_name="c", subcore_axis_name="s")

  @functools.partial(
      pl.kernel,
      out_type=(jax.ShapeDtypeStruct((_NB * 4 * _RP,), jnp.float32),
                jax.ShapeDtypeStruct((2 * 65536 + 32 * _PW,), jnp.int32)),
      mesh=mesh,
      scratch_types=[
          pltpu.VMEM((_BLK,), jnp.float32),
          pltpu.VMEM((_BLK,), jnp.int32),
          pltpu.VMEM((_BLK,), jnp.int32),
          pltpu.VMEM((_BLK,), jnp.int32),
          pltpu.VMEM((_CAPS // _NSUB,), jnp.int32),
          pltpu.VMEM((_CAPS // _NSUB,), jnp.int32),
          pltpu.VMEM((_BLK,), jnp.int32),
          pltpu.VMEM((_BLK,), jnp.int32),
          pltpu.VMEM((_BINS,), jnp.int32),
          pltpu.VMEM((128,), jnp.int32),
          pltpu.VMEM((128,), jnp.int32),
          pltpu.VMEM((128,), jnp.int32),
          pltpu.VMEM((128,), jnp.int32),
          pltpu.VMEM((128,), jnp.int32),
          pltpu.VMEM((_PW,), jnp.int32),
          pltpu.VMEM((_PW,), jnp.int32),
          pltpu.VMEM((_PW,), jnp.int32),
          pltpu.VMEM((_PW,), jnp.float32),
          pltpu.VMEM_SHARED((_CARR,), jnp.int32),
          pltpu.VMEM_SHARED((_CARR,), jnp.int32),
          pltpu.VMEM_SHARED((_BINS,), jnp.int32),
          pltpu.VMEM_SHARED((256,), jnp.int32),
          pltpu.SemaphoreType.DMA,
      ],
  )
  def k(mask_hbm, match_hbm, rand_hbm, cfg_hbm, out_hbm, ixh_hbm,
        fbuf, kv, iv, pv, ku_t, ix_t, pk, pi, hb, mbv, sidx, sval,
        gi, gv, rv, jv, av, fv, ku_c, ix_c, hist, meta, sem):
    cid = lax.axis_index("c")
    sid = lax.axis_index("s")
    i16 = lax.iota(jnp.int32, 16)
    zeros = jnp.zeros((16,), jnp.int32)

    pltpu.sync_copy(cfg_hbm, mbv)
    r_max = mbv[0:16][0]

    def batch_body(bi, carry0):
      batch = cid * 2 + bi
      moff = batch * _N
      shoff = sid * _SH

      def z(l, c):
        kv[_ds(l * 16, 16)] = zeros
        return c
      lax.fori_loop(0, _BLK // 16, z, jnp.int32(0))
      pltpu.sync_copy(kv.at[_ds(0, 1024)],
                      hist.at[_ds(sid * 1024, 1024)])

      @pl.when(sid == 0)
      def _():
        pltpu.sync_copy(kv.at[_ds(0, 256)], meta)
      plsc.subcore_barrier()

      def o(l, c):
        iv[_ds(l * 16, 16)] = jnp.full((16,), 1, jnp.int32)
        return c
      lax.fori_loop(0, _BLK // 16, o, jnp.int32(0))

      def blk1(t, c):
        pltpu.sync_copy(mask_hbm.at[_ds(moff + shoff + t * _BLK, _BLK)],
                        fbuf)
        def inner(l, c2):
          v = fbuf[_ds(l * 16, 16)]
          basc = jnp.minimum(jnp.maximum(v * 16384.0, 0.0), 16383.0)
          binv = 16383 - basc.astype(jnp.int32)
          pv[_ds(l * 16, 16)] = binv
          return c2
        lax.fori_loop(0, _BLK // 16, inner, jnp.int32(0))
        pltpu.sync_copy(iv, hist.at[pv], add=True)
        return c
      lax.fori_loop(0, _SH // _BLK, blk1, jnp.int32(0))
      plsc.subcore_barrier()

      @pl.when(sid == 0)
      def _():
        pltpu.sync_copy(hist, hb)
        def scan(j, carry):
          cum, bmax = carry
          h = hb[_ds(j * 16, 16)]
          inc = _prefix_incl(h) + jnp.broadcast_to(cum, (16,))
          newcum = inc[15]
          exceed = inc > jnp.broadcast_to(r_max, (16,))
          lane = _hmin16(jnp.where(exceed, i16, 16))[0]
          cand = j * 16 + lane
          found_now = jnp.logical_and(cum <= r_max, newcum > r_max)
          bmax = lax.select(found_now, cand, bmax)
          return (newcum, bmax)
        _, bmax = lax.fori_loop(0, _BINS // 16, scan,
                                (jnp.int32(0), jnp.int32(_BINS - 1)))
        def wsm(l, c):
          jj = l * 16 + i16
          sidx[_ds(l * 16, 16)] = jnp.where(jj == 0, 0, 128 + jj)
          sval[_ds(l * 16, 16)] = jnp.broadcast_to(bmax, (16,))
          return c
        lax.fori_loop(0, 8, wsm, jnp.int32(0))
        pltpu.sync_copy(sval, meta.at[sidx])
      plsc.subcore_barrier()

      pltpu.sync_copy(meta.at[_ds(0, 128)], mbv)
      b_max = mbv[0:16][0]

      def blk3(t, cvec):
        pltpu.sync_copy(mask_hbm.at[_ds(moff + shoff + t * _BLK, _BLK)],
                        fbuf)
        def inner(l, cv):
          v = fbuf[_ds(l * 16, 16)]
          basc = jnp.minimum(jnp.maximum(v * 16384.0, 0.0), 16383.0)
          binv = 16383 - basc.astype(jnp.int32)
          return cv + jnp.where(binv <= b_max, 1, 0)
        return lax.fori_loop(0, _BLK // 16, inner, cvec)
      cvec = lax.fori_loop(0, _SH // _BLK, blk3, zeros)
      cnt = _prefix_incl(cvec)[15]
      def wcm(l, c):
        jj = l * 16 + i16
        sidx[_ds(l * 16, 16)] = jnp.where(jj == 0, 16 + sid,
                                          128 + ((sid * 8 + jj) & 127))
        sval[_ds(l * 16, 16)] = jnp.broadcast_to(cnt, (16,))
        return c
      lax.fori_loop(0, 8, wcm, jnp.int32(0))
      pltpu.sync_copy(sval, meta.at[sidx])
      plsc.subcore_barrier()

      pltpu.sync_copy(meta.at[_ds(0, 128)], mbv)
      call = mbv[16:32]
      c128 = (call + 127) & (-128)
      pfx = _prefix_incl(c128)
      m128 = pfx[15]
      base_w = _dynlane(pfx - c128, sid)
      cnt_w = _dynlane(call, sid)
      c128_w = _dynlane(c128, sid)

      extra = (jnp.where(m128 > 65536, 1, 0)
               + jnp.where(m128 > 131072, 1, 0))
      pmax = 16 + extra
      n_dyn = lax.shift_left(jnp.int32(65536), extra)
      cs = n_dyn // _NSUB
      logl = pmax - 4
      trips = cs // _BLK

      def blk4(t, run):
        pltpu.sync_copy(mask_hbm.at[_ds(moff + shoff + t * _BLK, _BLK)],
                        fbuf)
        def inner(l, r):
          v = fbuf[_ds(l * 16, 16)]
          u = lax.bitcast_convert_type(v, jnp.int32)
          kms = jnp.where(u >= 0, u, u ^ 0x7FFFFFFF)
          kmd = ~kms
          basc = jnp.minimum(jnp.maximum(v * 16384.0, 0.0), 16383.0)
          binv = 16383 - basc.astype(jnp.int32)
          m = binv <= b_max
          mi = jnp.where(m, 1, 0)
          pfxi = _prefix_incl(mi)
          excl = pfxi - mi
          loc = t * _BLK + l * 16 + i16
          gidx = shoff + loc
          pos = jnp.where(m, base_w + r + excl,
                          _TRASH + (loc & (_BLK - 1)))
          kv[_ds(l * 16, 16)] = kmd
          iv[_ds(l * 16, 16)] = gidx
          pv[_ds(l * 16, 16)] = pos
          return r + pfxi[15]
        run = lax.fori_loop(0, _BLK // 16, inner, run)
        pltpu.sync_copy(kv, ku_c.at[pv])
        pltpu.sync_copy(iv, ix_c.at[pv])
        return run
      lax.fori_loop(0, _SH // _BLK, blk4, jnp.int32(0))

      gap = c128_w - cnt_w
      def gfill(l, c):
        j = l * 16 + i16
        gi[_ds(l * 16, 16)] = jnp.where(j < gap, base_w + cnt_w + j,
                                          _TRASH + j)
        gv[_ds(l * 16, 16)] = jnp.full((16,), _PADKEY, jnp.int32)
        return c
      lax.fori_loop(0, 8, gfill, jnp.int32(0))
      pltpu.sync_copy(gv, ku_c.at[gi])
      def gfill2(l, c):
        gv[_ds(l * 16, 16)] = jnp.full((16,), _PADIX, jnp.int32)
        return c
      lax.fori_loop(0, 8, gfill2, jnp.int32(0))
      pltpu.sync_copy(gv, ix_c.at[gi])

      def pfill(l, c):
        kv[_ds(l * 16, 16)] = jnp.full((16,), _PADKEY, jnp.int32)
        iv[_ds(l * 16, 16)] = jnp.full((16,), _PADIX, jnp.int32)
        return c
      lax.fori_loop(0, 8, pfill, jnp.int32(0))
      nfill = (n_dyn - m128) // 128
      mytrips = (nfill - sid + _NSUB - 1) // _NSUB
      def tfill(t, c):
        start = m128 + (sid + t * _NSUB) * 128
        pltpu.sync_copy(kv.at[_ds(0, 128)], ku_c.at[_ds(start, 128)])
        pltpu.sync_copy(iv.at[_ds(0, 128)], ix_c.at[_ds(start, 128)])
        return c
      lax.fori_loop(0, jnp.maximum(mytrips, 0), tfill, jnp.int32(0))
      plsc.subcore_barrier()

      def ld(t, c):
        pltpu.sync_copy(ku_c.at[_ds(sid * cs + t * _BLK, _BLK)],
                        ku_t.at[_ds(t * _BLK, _BLK)])
        pltpu.sync_copy(ix_c.at[_ds(sid * cs + t * _BLK, _BLK)],
                        ix_t.at[_ds(t * _BLK, _BLK)])
        return c
      lax.fori_loop(0, trips, ld, jnp.int32(0))

      def publish(t, c):
        pltpu.sync_copy(ku_t.at[_ds(t * _BLK, _BLK)],
                        ku_c.at[_ds(sid * cs + t * _BLK, _BLK)])
        pltpu.sync_copy(ix_t.at[_ds(t * _BLK, _BLK)],
                        ix_c.at[_ds(sid * cs + t * _BLK, _BLK)])
        return c

      def cross_stage(p, dlog):
        lax.fori_loop(0, trips, publish, jnp.int32(0))
        plsc.subcore_barrier()
        dchunks = lax.shift_left(jnp.int32(1), dlog - logl)
        partner = lax.bitwise_xor(sid, dchunks)
        kh_i = jnp.where((sid & dchunks) > 0, 1, 0)
        asc_i = 1 - (lax.shift_right_logical(sid * cs, p) & 1)
        tm_i = jnp.where(kh_i == asc_i, 1, 0)
        def cblk(t, c):
          pltpu.sync_copy(ku_c.at[_ds(partner * cs + t * _BLK, _BLK)], pk)
          pltpu.sync_copy(ix_c.at[_ds(partner * cs + t * _BLK, _BLK)], pi)
          @plsc.parallel_loop(0, _BLK // 16, 1, unroll=4)
          def ce(l):
            off = t * _BLK + l * 16
            a = ku_t[_ds(off, 16)]
            ai = ix_t[_ds(off, 16)]
            b = pk[_ds(l * 16, 16)]
            bi_ = pi[_ds(l * 16, 16)]
            m_i = jnp.where(_lex_gt(a, ai, b, bi_), 1, 0)
            tb = (m_i ^ jnp.broadcast_to(tm_i, (16,))) != 0
            ku_t[_ds(off, 16)] = jnp.where(tb, b, a)
            ix_t[_ds(off, 16)] = jnp.where(tb, bi_, ai)
          return c
        lax.fori_loop(0, trips, cblk, jnp.int32(0))
        plsc.subcore_barrier()

      def local_big_stage(p, dlog):
        d = lax.shift_left(jnp.int32(1), dlog)
        @plsc.parallel_loop(0, lax.shift_right_logical(cs, 5), 1, unroll=4)
        def bb(r):
          bidx = lax.shift_right_logical(r, dlog - 4)
          q = r & (lax.shift_right_logical(d, 4) - 1)
          boff = lax.shift_left(bidx, dlog + 1)
          off = boff + q * 16
          g0 = sid * cs + boff
          desc_i = lax.shift_right_logical(g0, p) & 1
          a = ku_t[_ds(off, 16)]
          ai = ix_t[_ds(off, 16)]
          b = ku_t[_ds(off + d, 16)]
          bi_ = ix_t[_ds(off + d, 16)]
          m_i = jnp.where(_lex_gt(a, ai, b, bi_), 1, 0)
          sw = (m_i ^ jnp.broadcast_to(desc_i, (16,))) != 0
          ku_t[_ds(off, 16)] = jnp.where(sw, b, a)
          ix_t[_ds(off, 16)] = jnp.where(sw, bi_, ai)
          ku_t[_ds(off + d, 16)] = jnp.where(sw, a, b)
          ix_t[_ds(off + d, 16)] = jnp.where(sw, ai, bi_)

      def invreg_stage(p, d):
        @plsc.parallel_loop(0, lax.shift_right_logical(cs, 4), 1, unroll=4)
        def qq(q):
          i16b = lax.iota(jnp.int32, 16)
          xor_idx = lax.bitwise_xor(i16b, d)
          il_i = jnp.where((i16b & d) == 0, 1, 0)
          off = q * 16
          g = sid * cs + off + i16b
          x = ku_t[_ds(off, 16)]
          xi = ix_t[_ds(off, 16)]
          px = _gather16(x, xor_idx)
          pxi = _gather16(xi, xor_idx)
          asc_i = 1 - (lax.shift_right_logical(
              g, jnp.broadcast_to(p, (16,))) & 1)
          m_i = jnp.where(_lex_gt(x, xi, px, pxi), 1, 0)
          ks_i = jnp.where(il_i == asc_i, 1, 0)
          tp = (m_i ^ ks_i ^ 1) != 0
          ku_t[_ds(off, 16)] = jnp.where(tp, px, x)
          ix_t[_ds(off, 16)] = jnp.where(tp, pxi, xi)

      def phase_body(p, c):
        def stage_body(t, c2):
          dlog = p - 1 - t
          @pl.when(dlog >= logl)
          def _():
            cross_stage(p, dlog)
          @pl.when(dlog < logl)
          def _():
            local_big_stage(p, dlog)
          return c2
        lax.fori_loop(0, jnp.maximum(p - 4, 0), stage_body, jnp.int32(0))
        for dlog in (3, 2, 1, 0):
          @pl.when(p > dlog)
          def _():
            invreg_stage(p, 1 << dlog)
        return c
      lax.fori_loop(1, pmax + 1, phase_body, jnp.int32(0))

      def fpub(t, c):
        pos = sid * cs + t * _BLK
        @pl.when(pos < 65536)
        def _():
          pltpu.sync_copy(ix_t.at[_ds(t * _BLK, _BLK)],
                          ixh_hbm.at[_ds(cid * 65536 + pos, _BLK)])
        return c
      lax.fori_loop(0, trips, fpub, jnp.int32(0))
      plsc.subcore_barrier()

      pltpu.sync_copy(rand_hbm.at[_ds(sid * _PW, _PW)], rv)
      ixview = ixh_hbm.at[_ds(cid * 65536, 65536)]
      for jj in range(_PW // 128):
        pltpu.sync_copy(ixview.at[rv.at[_ds(jj * 128, 128)]],
                        jv.at[_ds(jj * 128, 128)])
      jslot = 2 * 65536 + (cid * _NSUB + sid) * _PW
      pltpu.sync_copy(jv, ixh_hbm.at[_ds(jslot, _PW)])
      pltpu.sync_copy(ixh_hbm.at[_ds(jslot, _PW)], av)
      for ch in range(4):
        chview = match_hbm.at[_ds((batch * 4 + ch) * _N, _N)]
        for jj in range(_PW // 128):
          pltpu.sync_copy(chview.at[av.at[_ds(jj * 128, 128)]],
                          fv.at[_ds(jj * 128, 128)])
        pltpu.sync_copy(
            fv, out_hbm.at[_ds((batch * 4 + ch) * _RP + sid * _PW, _PW)])
      plsc.subcore_barrier()
      return carry0

    lax.fori_loop(0, 2, batch_body, jnp.int32(0))

  return k(mask_flat, match_flat, rand, cfg)


def kernel(match, mask):
  b = match.shape[0]
  n = match.shape[2] * match.shape[3]
  k = int(_RATIO * n)
  m_flat = match.reshape(-1)
  mask_flat = mask.reshape(-1)

  rand_int = jax.random.randint(jax.random.key(1), (_CHECK_NUM,), 0, k)
  rand_pad = jnp.concatenate(
      [rand_int.astype(jnp.int32),
       jnp.zeros((_RP - _CHECK_NUM,), jnp.int32)])
  r_max = jnp.max(rand_int).astype(jnp.int32)
  cfg = jnp.concatenate([r_max[None], jnp.zeros((127,), jnp.int32)])

  outf, _ = _sc_topk_sample(mask_flat, m_flat, rand_pad, cfg)
  return outf.reshape(b, 4, _RP)[:, :, :_CHECK_NUM]


# speedup vs baseline: 6.0390x; 1.0215x over previous
import functools

import jax
import jax.numpy as jnp
from jax import lax
from jax.experimental import pallas as pl
from jax.experimental.pallas import tpu as pltpu
from jax.experimental.pallas import tpu_sc as plsc

_CHECK_NUM = 6000
_RATIO = 0.2

_N = 262144
_NB = 4
_NSUB = 16
_SH = _N // _NSUB
_BINS = 16384
_BLK = 2048
_RP = 6144
_PW = _RP // _NSUB
_CAPS = 262144
_TRASH = _CAPS
_CARR = _CAPS + 4096
_PADKEY = 0x7FFFFFFF
_PADIX = 0x3FFFFFFF
_I16 = None


def _ds(off, size):
  if isinstance(off, int):
    return pl.ds(off, size)
  return pl.ds(pl.multiple_of(off, 8), size)


def _gather16(x, idx):
  dn = lax.GatherDimensionNumbers(
      offset_dims=(), collapsed_slice_dims=(0,), start_index_map=(0,))
  return lax.gather(x, idx[:, None], dn, (1,),
                    mode=lax.GatherScatterMode.PROMISE_IN_BOUNDS)


def _prefix_incl(x):
  i16 = lax.iota(jnp.int32, 16)
  acc = x
  for d in (1, 2, 4, 8):
    s = _gather16(acc, jnp.maximum(i16 - d, 0))
    acc = acc + jnp.where(i16 >= d, s, 0)
  return acc


def _hmin16(x):
  i16 = lax.iota(jnp.int32, 16)
  acc = x
  for d in (1, 2, 4, 8):
    s = _gather16(acc, jnp.minimum(i16 + d, 15))
    acc = jnp.minimum(acc, jnp.where(i16 < 16 - d, s, acc))
  return acc


def _dynlane(vec, s):
  i16 = lax.iota(jnp.int32, 16)
  return _gather16(vec, (i16 + s) & 15)[0]


def _bsel(cond_scalar, a, b):
  return jnp.where(jnp.broadcast_to(cond_scalar, (16,)), a, b)


def _lex_gt(ak, ai, bk, bi):
  return (ak > bk) | ((ak == bk) & (ai > bi))


def _sc_topk_sample(mask_flat, match_flat, rand, cfg):
  mesh = plsc.VectorSubcoreMesh(core_axis_name="c", subcore_axis_name="s")

  @functools.partial(
      pl.kernel,
      out_type=(jax.ShapeDtypeStruct((_NB * 4 * _RP,), jnp.float32),
                jax.ShapeDtypeStruct((2 * 65536 + 32 * _PW,), jnp.int32)),
      mesh=mesh,
      scratch_types=[
          pltpu.VMEM((_BLK,), jnp.float32),
          pltpu.VMEM((_BLK,), jnp.int32),
          pltpu.VMEM((_BLK,), jnp.int32),
          pltpu.VMEM((_BLK,), jnp.int32),
          pltpu.VMEM((_CAPS // _NSUB,), jnp.int32),
          pltpu.VMEM((_CAPS // _NSUB,), jnp.int32),
          pltpu.VMEM((_BLK,), jnp.int32),
          pltpu.VMEM((_BLK,), jnp.int32),
          pltpu.VMEM((_BINS,), jnp.int32),
          pltpu.VMEM((128,), jnp.int32),
          pltpu.VMEM((128,), jnp.int32),
          pltpu.VMEM((128,), jnp.int32),
          pltpu.VMEM((128,), jnp.int32),
          pltpu.VMEM((128,), jnp.int32),
          pltpu.VMEM((_PW,), jnp.int32),
          pltpu.VMEM((_PW,), jnp.int32),
          pltpu.VMEM((_PW,), jnp.int32),
          pltpu.VMEM((_PW,), jnp.float32),
          pltpu.VMEM_SHARED((_CARR,), jnp.int32),
          pltpu.VMEM_SHARED((_CARR,), jnp.int32),
          pltpu.VMEM_SHARED((_BINS,), jnp.int32),
          pltpu.VMEM_SHARED((256,), jnp.int32),
          pltpu.SemaphoreType.DMA,
      ],
  )
  def k(mask_hbm, match_hbm, rand_hbm, cfg_hbm, out_hbm, ixh_hbm,
        fbuf, kv, iv, pv, ku_t, ix_t, pk, pi, hb, mbv, sidx, sval,
        gi, gv, rv, jv, av, fv, ku_c, ix_c, hist, meta, sem):
    cid = lax.axis_index("c")
    sid = lax.axis_index("s")
    i16 = lax.iota(jnp.int32, 16)
    zeros = jnp.zeros((16,), jnp.int32)

    pltpu.sync_copy(cfg_hbm, mbv)
    r_max = mbv[0:16][0]

    def batch_body(bi, carry0):
      batch = cid * 2 + bi
      moff = batch * _N
      shoff = sid * _SH

      def z(l, c):
        kv[_ds(l * 16, 16)] = zeros
        return c
      lax.fori_loop(0, _BLK // 16, z, jnp.int32(0))
      pltpu.sync_copy(kv.at[_ds(0, 1024)],
                      hist.at[_ds(sid * 1024, 1024)])

      @pl.when(sid == 0)
      def _():
        pltpu.sync_copy(kv.at[_ds(0, 256)], meta)
      plsc.subcore_barrier()

      def o(l, c):
        iv[_ds(l * 16, 16)] = jnp.full((16,), 1, jnp.int32)
        return c
      lax.fori_loop(0, _BLK // 16, o, jnp.int32(0))

      def blk1(t, c):
        pltpu.sync_copy(mask_hbm.at[_ds(moff + shoff + t * _BLK, _BLK)],
                        fbuf)
        @plsc.parallel_loop(0, _BLK // 16, 1, unroll=8)
        def inner(l):
          v = fbuf[_ds(l * 16, 16)]
          basc = jnp.minimum(jnp.maximum(v * 16384.0, 0.0), 16383.0)
          binv = 16383 - basc.astype(jnp.int32)
          pv[_ds(l * 16, 16)] = binv
        pltpu.sync_copy(iv, hist.at[pv], add=True)
        return c
      lax.fori_loop(0, _SH // _BLK, blk1, jnp.int32(0))
      plsc.subcore_barrier()

      @pl.when(sid == 0)
      def _():
        pltpu.sync_copy(hist, hb)
        def scan(j, carry):
          cum, bmax = carry
          h = hb[_ds(j * 16, 16)]
          inc = _prefix_incl(h) + jnp.broadcast_to(cum, (16,))
          newcum = inc[15]
          exceed = inc > jnp.broadcast_to(r_max, (16,))
          lane = _hmin16(jnp.where(exceed, i16, 16))[0]
          cand = j * 16 + lane
          found_now = jnp.logical_and(cum <= r_max, newcum > r_max)
          bmax = lax.select(found_now, cand, bmax)
          return (newcum, bmax)
        _, bmax = lax.fori_loop(0, _BINS // 16, scan,
                                (jnp.int32(0), jnp.int32(_BINS - 1)))
        def wsm(l, c):
          jj = l * 16 + i16
          sidx[_ds(l * 16, 16)] = jnp.where(jj == 0, 0, 128 + jj)
          sval[_ds(l * 16, 16)] = jnp.broadcast_to(bmax, (16,))
          return c
        lax.fori_loop(0, 8, wsm, jnp.int32(0))
        pltpu.sync_copy(sval, meta.at[sidx])
      plsc.subcore_barrier()

      pltpu.sync_copy(meta.at[_ds(0, 128)], mbv)
      b_max = mbv[0:16][0]

      def blk3(t, cvec):
        pltpu.sync_copy(mask_hbm.at[_ds(moff + shoff + t * _BLK, _BLK)],
                        fbuf)
        def inner(l, cv):
          v = fbuf[_ds(l * 16, 16)]
          basc = jnp.minimum(jnp.maximum(v * 16384.0, 0.0), 16383.0)
          binv = 16383 - basc.astype(jnp.int32)
          return cv + jnp.where(binv <= b_max, 1, 0)
        return plsc.parallel_loop(0, _BLK // 16, 1, unroll=8,
                                  carry=cvec)(inner)
      cvec = lax.fori_loop(0, _SH // _BLK, blk3, zeros)
      cnt = _prefix_incl(cvec)[15]
      def wcm(l, c):
        jj = l * 16 + i16
        sidx[_ds(l * 16, 16)] = jnp.where(jj == 0, 16 + sid,
                                          128 + ((sid * 8 + jj) & 127))
        sval[_ds(l * 16, 16)] = jnp.broadcast_to(cnt, (16,))
        return c
      lax.fori_loop(0, 8, wcm, jnp.int32(0))
      pltpu.sync_copy(sval, meta.at[sidx])
      plsc.subcore_barrier()

      pltpu.sync_copy(meta.at[_ds(0, 128)], mbv)
      call = mbv[16:32]
      c128 = (call + 127) & (-128)
      pfx = _prefix_incl(c128)
      m128 = pfx[15]
      base_w = _dynlane(pfx - c128, sid)
      cnt_w = _dynlane(call, sid)
      c128_w = _dynlane(c128, sid)

      extra = (jnp.where(m128 > 65536, 1, 0)
               + jnp.where(m128 > 131072, 1, 0))
      pmax = 16 + extra
      n_dyn = lax.shift_left(jnp.int32(65536), extra)
      cs = n_dyn // _NSUB
      logl = pmax - 4
      trips = cs // _BLK

      def blk4(t, run):
        pltpu.sync_copy(mask_hbm.at[_ds(moff + shoff + t * _BLK, _BLK)],
                        fbuf)
        def inner(l, r):
          v = fbuf[_ds(l * 16, 16)]
          u = lax.bitcast_convert_type(v, jnp.int32)
          kms = jnp.where(u >= 0, u, u ^ 0x7FFFFFFF)
          kmd = ~kms
          basc = jnp.minimum(jnp.maximum(v * 16384.0, 0.0), 16383.0)
          binv = 16383 - basc.astype(jnp.int32)
          m = binv <= b_max
          mi = jnp.where(m, 1, 0)
          pfxi = _prefix_incl(mi)
          excl = pfxi - mi
          loc = t * _BLK + l * 16 + i16
          gidx = shoff + loc
          pos = jnp.where(m, base_w + r + excl,
                          _TRASH + (loc & (_BLK - 1)))
          kv[_ds(l * 16, 16)] = kmd
          iv[_ds(l * 16, 16)] = gidx
          pv[_ds(l * 16, 16)] = pos
          return r + pfxi[15]
        run = plsc.parallel_loop(0, _BLK // 16, 1, unroll=4,
                                 carry=run)(inner)
        pltpu.sync_copy(kv, ku_c.at[pv])
        pltpu.sync_copy(iv, ix_c.at[pv])
        return run
      lax.fori_loop(0, _SH // _BLK, blk4, jnp.int32(0))

      gap = c128_w - cnt_w
      def gfill(l, c):
        j = l * 16 + i16
        gi[_ds(l * 16, 16)] = jnp.where(j < gap, base_w + cnt_w + j,
                                          _TRASH + j)
        gv[_ds(l * 16, 16)] = jnp.full((16,), _PADKEY, jnp.int32)
        return c
      lax.fori_loop(0, 8, gfill, jnp.int32(0))
      pltpu.sync_copy(gv, ku_c.at[gi])
      def gfill2(l, c):
        gv[_ds(l * 16, 16)] = jnp.full((16,), _PADIX, jnp.int32)
        return c
      lax.fori_loop(0, 8, gfill2, jnp.int32(0))
      pltpu.sync_copy(gv, ix_c.at[gi])

      def pfill(l, c):
        kv[_ds(l * 16, 16)] = jnp.full((16,), _PADKEY, jnp.int32)
        iv[_ds(l * 16, 16)] = jnp.full((16,), _PADIX, jnp.int32)
        return c
      lax.fori_loop(0, 8, pfill, jnp.int32(0))
      nfill = (n_dyn - m128) // 128
      mytrips = (nfill - sid + _NSUB - 1) // _NSUB
      def tfill(t, c):
        start = m128 + (sid + t * _NSUB) * 128
        pltpu.sync_copy(kv.at[_ds(0, 128)], ku_c.at[_ds(start, 128)])
        pltpu.sync_copy(iv.at[_ds(0, 128)], ix_c.at[_ds(start, 128)])
        return c
      lax.fori_loop(0, jnp.maximum(mytrips, 0), tfill, jnp.int32(0))
      plsc.subcore_barrier()

      def ld(t, c):
        pltpu.sync_copy(ku_c.at[_ds(sid * cs + t * _BLK, _BLK)],
                        ku_t.at[_ds(t * _BLK, _BLK)])
        pltpu.sync_copy(ix_c.at[_ds(sid * cs + t * _BLK, _BLK)],
                        ix_t.at[_ds(t * _BLK, _BLK)])
        return c
      lax.fori_loop(0, trips, ld, jnp.int32(0))

      def publish(t, c):
        pltpu.sync_copy(ku_t.at[_ds(t * _BLK, _BLK)],
                        ku_c.at[_ds(sid * cs + t * _BLK, _BLK)])
        pltpu.sync_copy(ix_t.at[_ds(t * _BLK, _BLK)],
                        ix_c.at[_ds(sid * cs + t * _BLK, _BLK)])
        return c

      def cross_stage(p, dlog):
        lax.fori_loop(0, trips, publish, jnp.int32(0))
        plsc.subcore_barrier()
        dchunks = lax.shift_left(jnp.int32(1), dlog - logl)
        partner = lax.bitwise_xor(sid, dchunks)
        kh_i = jnp.where((sid & dchunks) > 0, 1, 0)
        asc_i = 1 - (lax.shift_right_logical(sid * cs, p) & 1)
        tm_i = jnp.where(kh_i == asc_i, 1, 0)
        def cblk(t, c):
          pltpu.sync_copy(ku_c.at[_ds(partner * cs + t * _BLK, _BLK)], pk)
          pltpu.sync_copy(ix_c.at[_ds(partner * cs + t * _BLK, _BLK)], pi)
          @plsc.parallel_loop(0, _BLK // 16, 1, unroll=8)
          def ce(l):
            off = t * _BLK + l * 16
            a = ku_t[_ds(off, 16)]
            ai = ix_t[_ds(off, 16)]
            b = pk[_ds(l * 16, 16)]
            bi_ = pi[_ds(l * 16, 16)]
            m_i = jnp.where(_lex_gt(a, ai, b, bi_), 1, 0)
            tb = (m_i ^ jnp.broadcast_to(tm_i, (16,))) != 0
            ku_t[_ds(off, 16)] = jnp.where(tb, b, a)
            ix_t[_ds(off, 16)] = jnp.where(tb, bi_, ai)
          return c
        lax.fori_loop(0, trips, cblk, jnp.int32(0))
        plsc.subcore_barrier()

      def local_big_stage(p, dlog):
        d = lax.shift_left(jnp.int32(1), dlog)
        @plsc.parallel_loop(0, lax.shift_right_logical(cs, 5), 1, unroll=8)
        def bb(r):
          bidx = lax.shift_right_logical(r, dlog - 4)
          q = r & (lax.shift_right_logical(d, 4) - 1)
          boff = lax.shift_left(bidx, dlog + 1)
          off = boff + q * 16
          g0 = sid * cs + boff
          desc_i = lax.shift_right_logical(g0, p) & 1
          a = ku_t[_ds(off, 16)]
          ai = ix_t[_ds(off, 16)]
          b = ku_t[_ds(off + d, 16)]
          bi_ = ix_t[_ds(off + d, 16)]
          m_i = jnp.where(_lex_gt(a, ai, b, bi_), 1, 0)
          sw = (m_i ^ jnp.broadcast_to(desc_i, (16,))) != 0
          ku_t[_ds(off, 16)] = jnp.where(sw, b, a)
          ix_t[_ds(off, 16)] = jnp.where(sw, bi_, ai)
          ku_t[_ds(off + d, 16)] = jnp.where(sw, a, b)
          ix_t[_ds(off + d, 16)] = jnp.where(sw, ai, bi_)

      def invreg_stage(p, d):
        @plsc.parallel_loop(0, lax.shift_right_logical(cs, 4), 1, unroll=8)
        def qq(q):
          i16b = lax.iota(jnp.int32, 16)
          xor_idx = lax.bitwise_xor(i16b, d)
          il_i = jnp.where((i16b & d) == 0, 1, 0)
          off = q * 16
          g = sid * cs + off + i16b
          x = ku_t[_ds(off, 16)]
          xi = ix_t[_ds(off, 16)]
          px = _gather16(x, xor_idx)
          pxi = _gather16(xi, xor_idx)
          asc_i = 1 - (lax.shift_right_logical(
              g, jnp.broadcast_to(p, (16,))) & 1)
          m_i = jnp.where(_lex_gt(x, xi, px, pxi), 1, 0)
          ks_i = jnp.where(il_i == asc_i, 1, 0)
          tp = (m_i ^ ks_i ^ 1) != 0
          ku_t[_ds(off, 16)] = jnp.where(tp, px, x)
          ix_t[_ds(off, 16)] = jnp.where(tp, pxi, xi)

      def phase_body(p, c):
        def stage_body(t, c2):
          dlog = p - 1 - t
          @pl.when(dlog >= logl)
          def _():
            cross_stage(p, dlog)
          @pl.when(dlog < logl)
          def _():
            local_big_stage(p, dlog)
          return c2
        lax.fori_loop(0, jnp.maximum(p - 4, 0), stage_body, jnp.int32(0))
        for dlog in (3, 2, 1, 0):
          @pl.when(p > dlog)
          def _():
            invreg_stage(p, 1 << dlog)
        return c
      lax.fori_loop(1, pmax + 1, phase_body, jnp.int32(0))

      def fpub(t, c):
        pos = sid * cs + t * _BLK
        @pl.when(pos < 65536)
        def _():
          pltpu.sync_copy(ix_t.at[_ds(t * _BLK, _BLK)],
                          ixh_hbm.at[_ds(cid * 65536 + pos, _BLK)])
        return c
      lax.fori_loop(0, trips, fpub, jnp.int32(0))
      plsc.subcore_barrier()

      pltpu.sync_copy(rand_hbm.at[_ds(sid * _PW, _PW)], rv)
      ixview = ixh_hbm.at[_ds(cid * 65536, 65536)]
      for jj in range(_PW // 128):
        pltpu.sync_copy(ixview.at[rv.at[_ds(jj * 128, 128)]],
                        jv.at[_ds(jj * 128, 128)])
      jslot = 2 * 65536 + (cid * _NSUB + sid) * _PW
      pltpu.sync_copy(jv, ixh_hbm.at[_ds(jslot, _PW)])
      pltpu.sync_copy(ixh_hbm.at[_ds(jslot, _PW)], av)
      for ch in range(4):
        chview = match_hbm.at[_ds((batch * 4 + ch) * _N, _N)]
        for jj in range(_PW // 128):
          pltpu.sync_copy(chview.at[av.at[_ds(jj * 128, 128)]],
                          fv.at[_ds(jj * 128, 128)])
        pltpu.sync_copy(
            fv, out_hbm.at[_ds((batch * 4 + ch) * _RP + sid * _PW, _PW)])
      plsc.subcore_barrier()
      return carry0

    lax.fori_loop(0, 2, batch_body, jnp.int32(0))

  return k(mask_flat, match_flat, rand, cfg)


def kernel(match, mask):
  b = match.shape[0]
  n = match.shape[2] * match.shape[3]
  k = int(_RATIO * n)
  m_flat = match.reshape(-1)
  mask_flat = mask.reshape(-1)

  rand_int = jax.random.randint(jax.random.key(1), (_CHECK_NUM,), 0, k)
  rand_pad = jnp.concatenate(
      [rand_int.astype(jnp.int32),
       jnp.zeros((_RP - _CHECK_NUM,), jnp.int32)])
  r_max = jnp.max(rand_int).astype(jnp.int32)
  cfg = jnp.concatenate([r_max[None], jnp.zeros((127,), jnp.int32)])

  outf, _ = _sc_topk_sample(mask_flat, m_flat, rand_pad, cfg)
  return outf.reshape(b, 4, _RP)[:, :, :_CHECK_NUM]
